# baseline (speedup 1.0000x reference)
import functools

import jax
import jax.numpy as jnp
from jax import lax
from jax.experimental import pallas as pl
from jax.experimental.pallas import tpu as pltpu
from jax.experimental.pallas import tpu_sc as plsc

N = 10000
E = 320000
IN_CH = 128
HID = 144
OUT_CH = 128
HEADS = 9
HEAD_DIM = 16
FF = 4 * HID
ROW = 160
SCALE = 0.25

NC = 2
NS = 16
NW = NC * NS
EW = E // NW
CB = 80
NCH = EW // CB
CB2 = 40
NCH2 = EW // CB2

BN = 1000
BE = 4000

_HI = lax.Precision.HIGHEST
_f32 = jnp.float32


def _dot(a, b):
    return jnp.dot(a, b, precision=_HI, preferred_element_type=_f32)


def _acos(x):
    ax = jnp.abs(x)
    p = jnp.float32(-0.0012624911)
    for c in (0.0066700901, -0.0170881256, 0.0308918810, -0.0501743046,
              0.0889789874, -0.2145988016, 1.5707963050):
        p = p * ax + jnp.float32(c)
    p = p * jnp.sqrt(jnp.maximum(1.0 - ax, 0.0))
    return jnp.where(x >= 0, p, jnp.float32(jnp.pi) - p)


def _lane_iota(n):
    return lax.broadcasted_iota(jnp.int32, (1, n), 1)


def _sel_chunk():
    r = lax.broadcasted_iota(jnp.int32, (HID, HEADS), 0)
    c = lax.broadcasted_iota(jnp.int32, (HID, HEADS), 1)
    return (r // HEAD_DIM == c).astype(_f32)


def _sel_bcast():
    r = lax.broadcasted_iota(jnp.int32, (HEADS, HID), 0)
    c = lax.broadcasted_iota(jnp.int32, (HEADS, HID), 1)
    return (c // HEAD_DIM == r).astype(_f32)


def _sel_block():
    r = lax.broadcasted_iota(jnp.int32, (HID, HID), 0)
    c = lax.broadcasted_iota(jnp.int32, (HID, HID), 1)
    return (r // HEAD_DIM == c // HEAD_DIM).astype(_f32)


def _ln(x, g, b):
    m = jnp.mean(x, axis=-1, keepdims=True)
    v = jnp.mean((x - m) * (x - m), axis=-1, keepdims=True)
    return (x - m) / jnp.sqrt(v + 1e-5) * g + b


def _emb_body(x_ref, w_ref, b_ref, o_ref):
    o_ref[...] = _dot(x_ref[...], w_ref[...]) + b_ref[...]


def _emb(x, wT, b):
    return pl.pallas_call(
        _emb_body,
        grid=(N // BN,),
        in_specs=[
            pl.BlockSpec((BN, IN_CH), lambda i: (i, 0)),
            pl.BlockSpec((IN_CH, HID), lambda i: (0, 0)),
            pl.BlockSpec((1, HID), lambda i: (0, 0)),
        ],
        out_specs=pl.BlockSpec((BN, HID), lambda i: (i, 0)),
        out_shape=jax.ShapeDtypeStruct((N, HID), _f32),
    )(x, wT, b)


def _qkv_body(h_ref, wq_ref, wk_ref, wv_ref, bq_ref, bk_ref, bv_ref,
              qt_ref, kt_ref, vt_ref):
    h = h_ref[...]
    B = h.shape[0]
    SS = _sel_block()
    S = _sel_chunk()
    l144 = _lane_iota(HID)

    n = jnp.sqrt(jnp.sum(h * h, axis=-1, keepdims=True) + 1e-12)
    un = jnp.clip(n, 1e-7, 1.0 - 1e-5)
    vh = (0.5 * jnp.log((1.0 + un) / (1.0 - un))) * h / jnp.maximum(n, 1e-7)
    xs = h / n
    d0 = jnp.clip(xs[:, HID - 1:HID], -1.0 + 1e-6, 1.0 - 1e-6)
    o143 = (l144 == HID - 1).astype(_f32)
    uu = xs - d0 * o143
    unorm = jnp.sqrt(jnp.sum(uu * uu, axis=-1, keepdims=True) + 1e-12)
    vs = _acos(d0) * uu / jnp.maximum(unorm, 1e-7)
    ve = h

    mhyp = l144 < 48
    meuc = (l144 >= 48) & (l144 < 96)
    o15 = (l144 % HEAD_DIM == HEAD_DIM - 1).astype(_f32)

    def mix(w, b):
        return jnp.concatenate(
            [_dot(vh, w[:, 0:48]), _dot(ve, w[:, 48:96]),
             _dot(vs, w[:, 96:144])], axis=-1) + b

    def expmap(pre):
        cn = jnp.sqrt(_dot(pre * pre, SS) + 1e-12)
        yh = jnp.tanh(cn) * pre / jnp.maximum(cn, 1e-7)
        yn = jnp.sqrt(_dot(yh * yh, SS) + 1e-12)
        oh = yh * jnp.minimum(1.0, (1.0 - 1e-5) / jnp.maximum(yn, 1e-7))
        ys = jnp.cos(cn) * o15 + jnp.sin(cn) * pre / jnp.maximum(cn, 1e-7)
        osph = ys / jnp.sqrt(_dot(ys * ys, SS) + 1e-12)
        return jnp.where(mhyp, oh, jnp.where(meuc, pre, osph))

    q = expmap(mix(wq_ref[...], bq_ref[...]))
    k = expmap(mix(wk_ref[...], bk_ref[...]))
    v = expmap(mix(wv_ref[...], bv_ref[...]))

    nq3 = _dot(q * q, S)[:, 0:3]
    nk3 = _dot(k * k, S)[:, 0:3]
    one3 = jnp.ones((B, 3), _f32)
    z10 = jnp.zeros((B, 10), _f32)
    qt_ref[...] = jnp.concatenate([q, nq3, one3, z10], axis=-1)
    kt_ref[...] = jnp.concatenate([k, one3, nk3, z10], axis=-1)
    vt_ref[...] = jnp.concatenate(
        [v, jnp.ones((B, HEADS), _f32), jnp.zeros((B, 7), _f32)], axis=-1)


def _qkv(h, wqT, wkT, wvT, bq, bk, bv):
    wspec = pl.BlockSpec((HID, HID), lambda i: (0, 0))
    bspec = pl.BlockSpec((1, HID), lambda i: (0, 0))
    tspec = pl.BlockSpec((BN, ROW), lambda i: (i, 0))
    tshape = jax.ShapeDtypeStruct((N, ROW), _f32)
    return pl.pallas_call(
        _qkv_body,
        grid=(N // BN,),
        in_specs=[pl.BlockSpec((BN, HID), lambda i: (i, 0)),
                  wspec, wspec, wspec, bspec, bspec, bspec],
        out_specs=(tspec, tspec, tspec),
        out_shape=(tshape, tshape, tshape),
    )(h, wqT, wkT, wvT, bq, bk, bv)


def _sc1_body(ridx3_ref, cidx3_ref, qt_ref, kt_ref, prod_ref,
              ridxv, cidxv, qb0, kb0, pb0, qb1, kb1, pb1,
              gsem0, gsem1, wsem0, wsem1):
    c = lax.axis_index("c")
    s = lax.axis_index("s")
    wid = s * NC + c
    base0 = wid * EW
    pltpu.sync_copy(ridx3_ref.at[wid], ridxv)
    pltpu.sync_copy(cidx3_ref.at[wid], cidxv)

    bufs = ((qb0, kb0, pb0, gsem0, wsem0), (qb1, kb1, pb1, gsem1, wsem1))

    def gpair(i, p):
        qb, kb, _, gsem, _ = bufs[p]
        gq = pltpu.make_async_copy(qt_ref.at[ridxv.at[i]], qb, gsem)
        gk = pltpu.make_async_copy(kt_ref.at[cidxv.at[i]], kb, gsem)
        return gq, gk

    def wb(i, p):
        _, _, pb, _, wsem = bufs[p]
        return pltpu.make_async_copy(
            pb, prod_ref.at[pl.ds(base0 + i * CB, CB)], wsem)

    for p in (0, 1):
        gq, gk = gpair(p, p)
        gq.start()
        gk.start()

    def chunk(i, carry):
        for p in (0, 1):

            @pl.when(i % 2 == p)
            def _():
                qb, kb, pb, gsem, wsem = bufs[p]
                gq, gk = gpair(i, p)
                gq.wait()
                gk.wait()

                @pl.when(i >= 2)
                def _():
                    wb(i - 2, p).wait()

                def mrow(j, carry2):
                    for t in range(ROW // 16):
                        sl = pl.ds(t * 16, 16)
                        pb[j, sl] = qb[j, sl] * kb[j, sl]
                    return carry2

                lax.fori_loop(0, CB, mrow, 0)

                @pl.when(i + 2 < NCH)
                def _():
                    g2q, g2k = gpair(i + 2, p)
                    g2q.start()
                    g2k.start()

                wb(i, p).start()
        return carry

    lax.fori_loop(0, NCH, chunk, 0)
    wb(NCH - 2, NCH % 2).wait()
    wb(NCH - 1, (NCH - 1) % 2).wait()


def _sc_pass1(ridx3, cidx3, qt, kt):
    call = pl.kernel(
        _sc1_body,
        out_type=jax.ShapeDtypeStruct((E, ROW), _f32),
        mesh=plsc.VectorSubcoreMesh(core_axis_name="c", subcore_axis_name="s",
                                    num_cores=NC, num_subcores=NS),
        compiler_params=pltpu.CompilerParams(use_tc_tiling_on_sc=False),
        scratch_types=[
            pltpu.VMEM((NCH, CB), jnp.int32),
            pltpu.VMEM((NCH, CB), jnp.int32),
            pltpu.VMEM((CB, ROW), _f32),
            pltpu.VMEM((CB, ROW), _f32),
            pltpu.VMEM((CB, ROW), _f32),
            pltpu.VMEM((CB, ROW), _f32),
            pltpu.VMEM((CB, ROW), _f32),
            pltpu.VMEM((CB, ROW), _f32),
            pltpu.SemaphoreType.DMA,
            pltpu.SemaphoreType.DMA,
            pltpu.SemaphoreType.DMA,
            pltpu.SemaphoreType.DMA,
        ],
    )
    return call(ridx3, cidx3, qt, kt)


def _logits_body(prod_ref, eexp_ref):
    prod = prod_ref[...]
    B = prod.shape[0]
    dots = _dot(prod[:, 0:HID], _sel_chunk())
    nq = prod[:, 144:147]
    nk = prod[:, 147:150]
    alpha = dots * SCALE
    d2 = nq + nk - 2.0 * dots[:, 0:3]
    den = jnp.maximum((1.0 - nq) * (1.0 - nk), 1e-7)
    z = jnp.maximum(1.0 + 2.0 * d2 / den, 1.0 + 1e-7)
    dh = jnp.log(z + jnp.sqrt(z * z - 1.0))
    t = jnp.clip(dots[:, 6:9], -1.0 + 1e-6, 1.0 - 1e-6)
    corr = jnp.concatenate(
        [0.1 * dh, jnp.zeros((B, 3), _f32), 0.1 * _acos(t)], axis=-1)
    e9 = jnp.exp(alpha - corr)
    e144 = _dot(e9, _sel_bcast())
    eexp_ref[...] = jnp.concatenate(
        [e144, e9, jnp.zeros((B, 7), _f32)], axis=-1)


def _logits(prod):
    return pl.pallas_call(
        _logits_body,
        grid=(E // BE,),
        in_specs=[pl.BlockSpec((BE, ROW), lambda i: (i, 0))],
        out_specs=pl.BlockSpec((BE, ROW), lambda i: (i, 0)),
        out_shape=jax.ShapeDtypeStruct((E, ROW), _f32),
    )(prod)


def _sc2_body(ridx2_ref, cidx2_ref, eexp_ref, vt_ref, zero_ref, out_ref,
              ridxv, cb0, cb1, vb0, vb1, eb, acc_sh,
              csem0, csem1, gsem0, gsem1, ssem0, ssem1):
    c = lax.axis_index("c")
    s = lax.axis_index("s")
    wid = s * NC + c
    base0 = wid * EW

    @pl.when(s == 0)
    def _zero():
        pltpu.sync_copy(zero_ref, acc_sh)

    plsc.subcore_barrier()
    pltpu.sync_copy(ridx2_ref.at[pl.ds(wid * NCH2, NCH2)], ridxv)

    cbufs = (cb0, cb1)
    vbufs = (vb0, vb1)
    csems = (csem0, csem1)
    gsems = (gsem0, gsem1)
    ssems = (ssem0, ssem1)

    def cidx_cp(i, p):
        return pltpu.make_async_copy(
            cidx2_ref.at[wid * NCH2 + i], cbufs[p], csems[p])

    def gath(p):
        return pltpu.make_async_copy(
            vt_ref.at[cbufs[p]], vbufs[p], gsems[p])

    def scat(i, p):
        return pltpu.make_async_copy(
            vbufs[p], acc_sh.at[ridxv.at[i]], ssems[p])

    cidx_cp(0, 0).start()
    cidx_cp(1, 1).start()
    cidx_cp(0, 0).wait()
    gath(0).start()

    def chunk(i, carry):
        for p in (0, 1):

            @pl.when(i % 2 == p)
            def _():
                gath(p).wait()

                @pl.when(i + 2 < NCH2)
                def _():
                    cidx_cp(i + 2, p).start()

                pltpu.sync_copy(eexp_ref.at[pl.ds(base0 + i * CB2, CB2)], eb)

                def mrow(j, carry2):
                    for t in range(ROW // 16):
                        sl = pl.ds(t * 16, 16)
                        vbufs[p][j, sl] = eb[j, sl] * vbufs[p][j, sl]
                    return carry2

                lax.fori_loop(0, CB2, mrow, 0)
                pltpu.async_copy(vbufs[p], acc_sh.at[ridxv.at[i]],
                                 ssems[p], add=True)

                @pl.when(i >= 1)
                def _():
                    scat(i - 1, 1 - p).wait()

                @pl.when(i + 1 < NCH2)
                def _():
                    cidx_cp(i + 1, 1 - p).wait()
                    gath(1 - p).start()
        return carry

    lax.fori_loop(0, NCH2, chunk, 0)
    scat(NCH2 - 1, (NCH2 - 1) % 2).wait()
    plsc.subcore_barrier()

    @pl.when(s == 0)
    def _flush():
        pltpu.sync_copy(acc_sh, out_ref.at[c])


def _sc_pass2(ridx2, cidx2, eexp, vt, zeros_acc):
    call = pl.kernel(
        _sc2_body,
        out_type=jax.ShapeDtypeStruct((NC, N, ROW), _f32),
        mesh=plsc.VectorSubcoreMesh(core_axis_name="c", subcore_axis_name="s",
                                    num_cores=NC, num_subcores=NS),
        compiler_params=pltpu.CompilerParams(use_tc_tiling_on_sc=False),
        scratch_types=[
            pltpu.VMEM((NCH2, CB2), jnp.int32),
            pltpu.VMEM((CB2,), jnp.int32),
            pltpu.VMEM((CB2,), jnp.int32),
            pltpu.VMEM((CB2, ROW), _f32),
            pltpu.VMEM((CB2, ROW), _f32),
            pltpu.VMEM((CB2, ROW), _f32),
            pltpu.VMEM_SHARED((N, ROW), _f32),
            pltpu.SemaphoreType.DMA,
            pltpu.SemaphoreType.DMA,
            pltpu.SemaphoreType.DMA,
            pltpu.SemaphoreType.DMA,
            pltpu.SemaphoreType.DMA,
            pltpu.SemaphoreType.DMA,
        ],
    )
    return call(ridx2, cidx2, eexp, vt, zeros_acc)


def _post_body(h_ref, a0_ref, a1_ref, lo_ref, lob_ref, f1_ref, f1b_ref,
               f2_ref, f2b_ref, g1_ref, b1_ref, g2_ref, b2_ref, out_ref):
    acc = a0_ref[...] + a1_ref[...]
    s144 = _dot(acc[:, 144:144 + HEADS], _sel_bcast())
    att = acc[:, 0:HID] / (s144 + 1e-16)
    att = _dot(att, lo_ref[...]) + lob_ref[...]
    h1 = _ln(h_ref[...] + att, g1_ref[...], b1_ref[...])
    ffp = _dot(h1, f1_ref[...]) + f1b_ref[...]
    gl = ffp * 0.5 * (1.0 + lax.erf(ffp * jnp.float32(0.7071067811865475)))
    ff = _dot(gl, f2_ref[...]) + f2b_ref[...]
    out_ref[...] = _ln(h1 + ff, g2_ref[...], b2_ref[...])


def _post(h, a0, a1, loT, lob, f1T, f1b, f2T, f2b, g1, b1, g2, b2):
    vspec = pl.BlockSpec((1, HID), lambda i: (0, 0))
    return pl.pallas_call(
        _post_body,
        grid=(N // BN,),
        in_specs=[
            pl.BlockSpec((BN, HID), lambda i: (i, 0)),
            pl.BlockSpec((BN, ROW), lambda i: (i, 0)),
            pl.BlockSpec((BN, ROW), lambda i: (i, 0)),
            pl.BlockSpec((HID, HID), lambda i: (0, 0)), vspec,
            pl.BlockSpec((HID, FF), lambda i: (0, 0)),
            pl.BlockSpec((1, FF), lambda i: (0, 0)),
            pl.BlockSpec((FF, HID), lambda i: (0, 0)), vspec,
            vspec, vspec, vspec, vspec,
        ],
        out_specs=pl.BlockSpec((BN, HID), lambda i: (i, 0)),
        out_shape=jax.ShapeDtypeStruct((N, HID), _f32),
    )(h, a0, a1, loT, lob, f1T, f1b, f2T, f2b, g1, b1, g2, b2)


def _final_body(h_ref, w_ref, b_ref, o_ref):
    o_ref[...] = _dot(h_ref[...], w_ref[...]) + b_ref[...]


def _final(h, wT, b):
    return pl.pallas_call(
        _final_body,
        grid=(N // BN,),
        in_specs=[
            pl.BlockSpec((BN, HID), lambda i: (i, 0)),
            pl.BlockSpec((HID, OUT_CH), lambda i: (0, 0)),
            pl.BlockSpec((1, OUT_CH), lambda i: (0, 0)),
        ],
        out_specs=pl.BlockSpec((BN, OUT_CH), lambda i: (i, 0)),
        out_shape=jax.ShapeDtypeStruct((N, OUT_CH), _f32),
    )(h, wT, b)


def kernel(x, edge_index, params):
    row = edge_index[0]
    col = edge_index[1]
    r3 = row.reshape(NW, NCH, CB)
    c3 = col.reshape(NW, NCH, CB)
    r2 = row.reshape(NW * NCH2, CB2)
    c2 = col.reshape(NW * NCH2, CB2)
    zeros_acc = jnp.zeros((N, ROW), _f32)

    h = _emb(x, params['emb_W'].T, params['emb_b'].reshape(1, -1))
    for lp in params['layers']:
        heads = lp['heads']
        wqT = jnp.concatenate([hp['Wq'].T for hp in heads], axis=1)
        wkT = jnp.concatenate([hp['Wk'].T for hp in heads], axis=1)
        wvT = jnp.concatenate([hp['Wv'].T for hp in heads], axis=1)
        bq = jnp.concatenate([hp['bq'] for hp in heads]).reshape(1, -1)
        bk = jnp.concatenate([hp['bk'] for hp in heads]).reshape(1, -1)
        bv = jnp.concatenate([hp['bv'] for hp in heads]).reshape(1, -1)

        qt, kt, vt = _qkv(h, wqT, wkT, wvT, bq, bk, bv)
        prod = _sc_pass1(r3, c3, qt, kt)
        eexp = _logits(prod)
        accs = _sc_pass2(r2, c2, eexp, vt, zeros_acc)
        h = _post(h, accs[0], accs[1],
                  lp['lo_W'].T, lp['lo_b'].reshape(1, -1),
                  lp['f1_W'].T, lp['f1_b'].reshape(1, -1),
                  lp['f2_W'].T, lp['f2_b'].reshape(1, -1),
                  lp['ln1_g'].reshape(1, -1), lp['ln1_b'].reshape(1, -1),
                  lp['ln2_g'].reshape(1, -1), lp['ln2_b'].reshape(1, -1))

    return _final(h, params['out_W'].T, params['out_b'].reshape(1, -1))


# speedup vs baseline: 1.2672x; 1.2672x over previous
import functools

import jax
import jax.numpy as jnp
from jax import lax
from jax.experimental import pallas as pl
from jax.experimental.pallas import tpu as pltpu
from jax.experimental.pallas import tpu_sc as plsc

N = 10000
E = 320000
IN_CH = 128
HID = 144
OUT_CH = 128
HEADS = 9
HEAD_DIM = 16
FF = 4 * HID
ROW = 160
ROWQ = 256
SCALE = 0.25

NC = 2
NS = 16
NW = NC * NS
EW = E // NW
CB = 80
NCH = EW // CB
CB2 = 40
NCH2 = EW // CB2

BN = 1000
BE = 4000

_HI = lax.Precision.HIGHEST
_f32 = jnp.float32


def _dot(a, b):
    return jnp.dot(a, b, precision=_HI, preferred_element_type=_f32)


def _acos(x):
    ax = jnp.abs(x)
    p = jnp.float32(-0.0012624911)
    for c in (0.0066700901, -0.0170881256, 0.0308918810, -0.0501743046,
              0.0889789874, -0.2145988016, 1.5707963050):
        p = p * ax + jnp.float32(c)
    p = p * jnp.sqrt(jnp.maximum(1.0 - ax, 0.0))
    return jnp.where(x >= 0, p, jnp.float32(jnp.pi) - p)


def _lane_iota(n):
    return lax.broadcasted_iota(jnp.int32, (1, n), 1)


def _sel_chunk():
    r = lax.broadcasted_iota(jnp.int32, (HID, HEADS), 0)
    c = lax.broadcasted_iota(jnp.int32, (HID, HEADS), 1)
    return (r // HEAD_DIM == c).astype(_f32)


def _sel_bcast():
    r = lax.broadcasted_iota(jnp.int32, (HEADS, HID), 0)
    c = lax.broadcasted_iota(jnp.int32, (HEADS, HID), 1)
    return (c // HEAD_DIM == r).astype(_f32)


def _sel_block():
    r = lax.broadcasted_iota(jnp.int32, (HID, HID), 0)
    c = lax.broadcasted_iota(jnp.int32, (HID, HID), 1)
    return (r // HEAD_DIM == c // HEAD_DIM).astype(_f32)


def _ln(x, g, b):
    m = jnp.mean(x, axis=-1, keepdims=True)
    v = jnp.mean((x - m) * (x - m), axis=-1, keepdims=True)
    return (x - m) / jnp.sqrt(v + 1e-5) * g + b


def _emb_body(x_ref, w_ref, b_ref, o_ref):
    o_ref[...] = _dot(x_ref[...], w_ref[...]) + b_ref[...]


def _emb(x, wT, b):
    return pl.pallas_call(
        _emb_body,
        grid=(N // BN,),
        in_specs=[
            pl.BlockSpec((BN, IN_CH), lambda i: (i, 0)),
            pl.BlockSpec((IN_CH, HID), lambda i: (0, 0)),
            pl.BlockSpec((1, HID), lambda i: (0, 0)),
        ],
        out_specs=pl.BlockSpec((BN, HID), lambda i: (i, 0)),
        out_shape=jax.ShapeDtypeStruct((N, HID), _f32),
    )(x, wT, b)


def _qkv_body(h_ref, wq_ref, wk_ref, wv_ref, bq_ref, bk_ref, bv_ref,
              qt_ref, kt_ref, vt_ref):
    h = h_ref[...]
    B = h.shape[0]
    SS = _sel_block()
    S = _sel_chunk()
    l144 = _lane_iota(HID)

    def csum(x):
        return _dot(x, SS)

    def csum9(x):
        return _dot(x, S)

    n = jnp.sqrt(jnp.sum(h * h, axis=-1, keepdims=True) + 1e-12)
    un = jnp.clip(n, 1e-7, 1.0 - 1e-5)
    vh = (0.5 * jnp.log((1.0 + un) / (1.0 - un))) * h / jnp.maximum(n, 1e-7)
    xs = h / n
    d0 = jnp.clip(xs[:, HID - 1:HID], -1.0 + 1e-6, 1.0 - 1e-6)
    o143 = (l144 == HID - 1).astype(_f32)
    uu = xs - d0 * o143
    unorm = jnp.sqrt(jnp.sum(uu * uu, axis=-1, keepdims=True) + 1e-12)
    vs = _acos(d0) * uu / jnp.maximum(unorm, 1e-7)
    ve = h

    mhyp = l144 < 48
    meuc = (l144 >= 48) & (l144 < 96)
    o15 = (l144 % HEAD_DIM == HEAD_DIM - 1).astype(_f32)

    def mix(w, b):
        return jnp.concatenate(
            [_dot(vh, w[:, 0:48]), _dot(ve, w[:, 48:96]),
             _dot(vs, w[:, 96:144])], axis=-1) + b

    def expmap(pre):
        cn = jnp.sqrt(csum(pre * pre) + 1e-12)
        yh = jnp.tanh(cn) * pre / jnp.maximum(cn, 1e-7)
        yn = jnp.sqrt(csum(yh * yh) + 1e-12)
        oh = yh * jnp.minimum(1.0, (1.0 - 1e-5) / jnp.maximum(yn, 1e-7))
        ys = jnp.cos(cn) * o15 + jnp.sin(cn) * pre / jnp.maximum(cn, 1e-7)
        osph = ys / jnp.sqrt(csum(ys * ys) + 1e-12)
        return jnp.where(mhyp, oh, jnp.where(meuc, pre, osph))

    q = expmap(mix(wq_ref[...], bq_ref[...]))
    k = expmap(mix(wk_ref[...], bk_ref[...]))
    v = expmap(mix(wv_ref[...], bv_ref[...]))

    nq3 = csum9(q * q)[:, 0:3]
    nk3 = csum9(k * k)[:, 0:3]
    one3 = jnp.ones((B, 3), _f32)

    rr = lax.broadcasted_iota(jnp.int32, (HID, ROWQ), 0)
    cc = lax.broadcasted_iota(jnp.int32, (HID, ROWQ), 1)
    P = (((cc % 16) == (rr // 16)) & ((cc // 16) == (rr % 16))).astype(_f32)

    def esel(off):
        r3 = lax.broadcasted_iota(jnp.int32, (3, ROWQ), 0)
        c3 = lax.broadcasted_iota(jnp.int32, (3, ROWQ), 1)
        return (c3 == off + r3).astype(_f32)

    e0 = esel(9)
    e1 = esel(25)
    e2 = esel(44)
    qt_ref[...] = (_dot(q, P) + _dot(nq3, e0) + _dot(one3, e1)
                   + _dot(1.0 - nq3, e2))
    kt_ref[...] = (_dot(k, P) + _dot(one3, e0) + _dot(nk3, e1)
                   + _dot(1.0 - nk3, e2))
    vt_ref[...] = jnp.concatenate(
        [v, jnp.ones((B, HEADS), _f32), jnp.zeros((B, 7), _f32)], axis=-1)


def _qkv(h, wqT, wkT, wvT, bq, bk, bv):
    wspec = pl.BlockSpec((HID, HID), lambda i: (0, 0))
    bspec = pl.BlockSpec((1, HID), lambda i: (0, 0))
    qspec = pl.BlockSpec((BN, ROWQ), lambda i: (i, 0))
    qshape = jax.ShapeDtypeStruct((N, ROWQ), _f32)
    return pl.pallas_call(
        _qkv_body,
        grid=(N // BN,),
        in_specs=[pl.BlockSpec((BN, HID), lambda i: (i, 0)),
                  wspec, wspec, wspec, bspec, bspec, bspec],
        out_specs=(qspec, qspec,
                   pl.BlockSpec((BN, ROW), lambda i: (i, 0))),
        out_shape=(qshape, qshape,
                   jax.ShapeDtypeStruct((N, ROW), _f32)),
    )(h, wqT, wkT, wvT, bq, bk, bv)


def _sc1_body(ridx3_ref, cidx3_ref, qt_ref, kt_ref, prod_ref,
              ridxv, cidxv, qb0, kb0, pb0, qb1, kb1, pb1,
              gsem0, gsem1, wsem0, wsem1):
    c = lax.axis_index("c")
    s = lax.axis_index("s")
    wid = s * NC + c
    base0 = wid * EW
    pltpu.sync_copy(ridx3_ref.at[wid], ridxv)
    pltpu.sync_copy(cidx3_ref.at[wid], cidxv)

    bufs = ((qb0, kb0, pb0, gsem0, wsem0), (qb1, kb1, pb1, gsem1, wsem1))

    def gpair(i, p):
        qb, kb, _, gsem, _ = bufs[p]
        gq = pltpu.make_async_copy(qt_ref.at[ridxv.at[i]], qb, gsem)
        gk = pltpu.make_async_copy(kt_ref.at[cidxv.at[i]], kb, gsem)
        return gq, gk

    def wb(i, p):
        _, _, pb, _, wsem = bufs[p]
        return pltpu.make_async_copy(
            pb, prod_ref.at[pl.ds(base0 + i * CB, CB)], wsem)

    for p in (0, 1):
        gq, gk = gpair(p, p)
        gq.start()
        gk.start()

    def chunk(i, carry):
        for p in (0, 1):

            @pl.when(i % 2 == p)
            def _():
                qb, kb, pb, gsem, wsem = bufs[p]
                gq, gk = gpair(i, p)
                gq.wait()
                gk.wait()

                @pl.when(i >= 2)
                def _():
                    wb(i - 2, p).wait()

                def mrow(j, carry2):
                    acc = qb[j, pl.ds(0, 16)] * kb[j, pl.ds(0, 16)]
                    for t in range(1, ROWQ // 16):
                        sl = pl.ds(t * 16, 16)
                        acc = acc + qb[j, sl] * kb[j, sl]
                    pb[j, pl.ds(0, 16)] = acc
                    return carry2

                lax.fori_loop(0, CB, mrow, 0)

                @pl.when(i + 2 < NCH)
                def _():
                    g2q, g2k = gpair(i + 2, p)
                    g2q.start()
                    g2k.start()

                wb(i, p).start()
        return carry

    lax.fori_loop(0, NCH, chunk, 0)
    wb(NCH - 2, NCH % 2).wait()
    wb(NCH - 1, (NCH - 1) % 2).wait()


def _sc_pass1(ridx3, cidx3, qt, kt):
    call = pl.kernel(
        _sc1_body,
        out_type=jax.ShapeDtypeStruct((E, 16), _f32),
        mesh=plsc.VectorSubcoreMesh(core_axis_name="c", subcore_axis_name="s",
                                    num_cores=NC, num_subcores=NS),
        compiler_params=pltpu.CompilerParams(use_tc_tiling_on_sc=False),
        scratch_types=[
            pltpu.VMEM((NCH, CB), jnp.int32),
            pltpu.VMEM((NCH, CB), jnp.int32),
            pltpu.VMEM((CB, ROWQ), _f32),
            pltpu.VMEM((CB, ROWQ), _f32),
            pltpu.VMEM((CB, 16), _f32),
            pltpu.VMEM((CB, ROWQ), _f32),
            pltpu.VMEM((CB, ROWQ), _f32),
            pltpu.VMEM((CB, 16), _f32),
            pltpu.SemaphoreType.DMA,
            pltpu.SemaphoreType.DMA,
            pltpu.SemaphoreType.DMA,
            pltpu.SemaphoreType.DMA,
        ],
    )
    return call(ridx3, cidx3, qt, kt)


def _logits_body(prod_ref, eexp_ref):
    prod = prod_ref[...]
    B = prod.shape[0]
    dots = prod[:, 0:HEADS]
    s3 = prod[:, 9:12]
    den3 = prod[:, 12:15]
    alpha = dots * SCALE
    d2 = s3 - 2.0 * dots[:, 0:3]
    den = jnp.maximum(den3, 1e-7)
    z = jnp.maximum(1.0 + 2.0 * d2 / den, 1.0 + 1e-7)
    dh = jnp.log(z + jnp.sqrt(z * z - 1.0))
    t = jnp.clip(dots[:, 6:9], -1.0 + 1e-6, 1.0 - 1e-6)
    corr = jnp.concatenate(
        [0.1 * dh, jnp.zeros((B, 3), _f32), 0.1 * _acos(t)], axis=-1)
    e9 = jnp.exp(alpha - corr)
    e144 = _dot(e9, _sel_bcast())
    eexp_ref[...] = jnp.concatenate(
        [e144, e9, jnp.zeros((B, 7), _f32)], axis=-1)


def _logits(prod):
    return pl.pallas_call(
        _logits_body,
        grid=(E // BE,),
        in_specs=[pl.BlockSpec((BE, 16), lambda i: (i, 0))],
        out_specs=pl.BlockSpec((BE, ROW), lambda i: (i, 0)),
        out_shape=jax.ShapeDtypeStruct((E, ROW), _f32),
    )(prod)


def _sc2_body(ridx2_ref, cidx2_ref, eexp_ref, vt_ref, zero_ref, out_ref,
              ridxv, cb0, cb1, vb0, vb1, eb, acc_sh,
              csem0, csem1, gsem0, gsem1, ssem0, ssem1):
    c = lax.axis_index("c")
    s = lax.axis_index("s")
    wid = s * NC + c
    base0 = wid * EW

    @pl.when(s == 0)
    def _zero():
        pltpu.sync_copy(zero_ref, acc_sh)

    plsc.subcore_barrier()
    pltpu.sync_copy(ridx2_ref.at[pl.ds(wid * NCH2, NCH2)], ridxv)

    cbufs = (cb0, cb1)
    vbufs = (vb0, vb1)
    csems = (csem0, csem1)
    gsems = (gsem0, gsem1)
    ssems = (ssem0, ssem1)

    def cidx_cp(i, p):
        return pltpu.make_async_copy(
            cidx2_ref.at[wid * NCH2 + i], cbufs[p], csems[p])

    def gath(p):
        return pltpu.make_async_copy(
            vt_ref.at[cbufs[p]], vbufs[p], gsems[p])

    def scat(i, p):
        return pltpu.make_async_copy(
            vbufs[p], acc_sh.at[ridxv.at[i]], ssems[p])

    cidx_cp(0, 0).start()
    cidx_cp(1, 1).start()
    cidx_cp(0, 0).wait()
    gath(0).start()

    def chunk(i, carry):
        for p in (0, 1):

            @pl.when(i % 2 == p)
            def _():
                gath(p).wait()

                @pl.when(i + 2 < NCH2)
                def _():
                    cidx_cp(i + 2, p).start()

                pltpu.sync_copy(eexp_ref.at[pl.ds(base0 + i * CB2, CB2)], eb)

                def mrow(j, carry2):
                    for t in range(ROW // 16):
                        sl = pl.ds(t * 16, 16)
                        vbufs[p][j, sl] = eb[j, sl] * vbufs[p][j, sl]
                    return carry2

                lax.fori_loop(0, CB2, mrow, 0)
                pltpu.async_copy(vbufs[p], acc_sh.at[ridxv.at[i]],
                                 ssems[p], add=True)

                @pl.when(i >= 1)
                def _():
                    scat(i - 1, 1 - p).wait()

                @pl.when(i + 1 < NCH2)
                def _():
                    cidx_cp(i + 1, 1 - p).wait()
                    gath(1 - p).start()
        return carry

    lax.fori_loop(0, NCH2, chunk, 0)
    scat(NCH2 - 1, (NCH2 - 1) % 2).wait()
    plsc.subcore_barrier()

    @pl.when(s == 0)
    def _flush():
        pltpu.sync_copy(acc_sh, out_ref.at[c])


def _sc_pass2(ridx2, cidx2, eexp, vt, zeros_acc):
    call = pl.kernel(
        _sc2_body,
        out_type=jax.ShapeDtypeStruct((NC, N, ROW), _f32),
        mesh=plsc.VectorSubcoreMesh(core_axis_name="c", subcore_axis_name="s",
                                    num_cores=NC, num_subcores=NS),
        compiler_params=pltpu.CompilerParams(use_tc_tiling_on_sc=False),
        scratch_types=[
            pltpu.VMEM((NCH2, CB2), jnp.int32),
            pltpu.VMEM((CB2,), jnp.int32),
            pltpu.VMEM((CB2,), jnp.int32),
            pltpu.VMEM((CB2, ROW), _f32),
            pltpu.VMEM((CB2, ROW), _f32),
            pltpu.VMEM((CB2, ROW), _f32),
            pltpu.VMEM_SHARED((N, ROW), _f32),
            pltpu.SemaphoreType.DMA,
            pltpu.SemaphoreType.DMA,
            pltpu.SemaphoreType.DMA,
            pltpu.SemaphoreType.DMA,
            pltpu.SemaphoreType.DMA,
            pltpu.SemaphoreType.DMA,
        ],
    )
    return call(ridx2, cidx2, eexp, vt, zeros_acc)


def _post_body(h_ref, a0_ref, a1_ref, lo_ref, lob_ref, f1_ref, f1b_ref,
               f2_ref, f2b_ref, g1_ref, b1_ref, g2_ref, b2_ref, out_ref):
    acc = a0_ref[...] + a1_ref[...]
    s144 = _dot(acc[:, 144:144 + HEADS], _sel_bcast())
    att = acc[:, 0:HID] / (s144 + 1e-16)
    att = _dot(att, lo_ref[...]) + lob_ref[...]
    h1 = _ln(h_ref[...] + att, g1_ref[...], b1_ref[...])
    ffp = _dot(h1, f1_ref[...]) + f1b_ref[...]
    gl = ffp * 0.5 * (1.0 + lax.erf(ffp * jnp.float32(0.7071067811865475)))
    ff = _dot(gl, f2_ref[...]) + f2b_ref[...]
    out_ref[...] = _ln(h1 + ff, g2_ref[...], b2_ref[...])


def _post(h, a0, a1, loT, lob, f1T, f1b, f2T, f2b, g1, b1, g2, b2):
    vspec = pl.BlockSpec((1, HID), lambda i: (0, 0))
    return pl.pallas_call(
        _post_body,
        grid=(N // BN,),
        in_specs=[
            pl.BlockSpec((BN, HID), lambda i: (i, 0)),
            pl.BlockSpec((BN, ROW), lambda i: (i, 0)),
            pl.BlockSpec((BN, ROW), lambda i: (i, 0)),
            pl.BlockSpec((HID, HID), lambda i: (0, 0)), vspec,
            pl.BlockSpec((HID, FF), lambda i: (0, 0)),
            pl.BlockSpec((1, FF), lambda i: (0, 0)),
            pl.BlockSpec((FF, HID), lambda i: (0, 0)), vspec,
            vspec, vspec, vspec, vspec,
        ],
        out_specs=pl.BlockSpec((BN, HID), lambda i: (i, 0)),
        out_shape=jax.ShapeDtypeStruct((N, HID), _f32),
    )(h, a0, a1, loT, lob, f1T, f1b, f2T, f2b, g1, b1, g2, b2)


def _final_body(h_ref, w_ref, b_ref, o_ref):
    o_ref[...] = _dot(h_ref[...], w_ref[...]) + b_ref[...]


def _final(h, wT, b):
    return pl.pallas_call(
        _final_body,
        grid=(N // BN,),
        in_specs=[
            pl.BlockSpec((BN, HID), lambda i: (i, 0)),
            pl.BlockSpec((HID, OUT_CH), lambda i: (0, 0)),
            pl.BlockSpec((1, OUT_CH), lambda i: (0, 0)),
        ],
        out_specs=pl.BlockSpec((BN, OUT_CH), lambda i: (i, 0)),
        out_shape=jax.ShapeDtypeStruct((N, OUT_CH), _f32),
    )(h, wT, b)


def kernel(x, edge_index, params):
    row = edge_index[0]
    col = edge_index[1]
    r3 = row.reshape(NW, NCH, CB)
    c3 = col.reshape(NW, NCH, CB)
    r2 = row.reshape(NW * NCH2, CB2)
    c2 = col.reshape(NW * NCH2, CB2)
    zeros_acc = jnp.zeros((N, ROW), _f32)

    h = _emb(x, params['emb_W'].T, params['emb_b'].reshape(1, -1))
    for lp in params['layers']:
        heads = lp['heads']
        wqT = jnp.concatenate([hp['Wq'].T for hp in heads], axis=1)
        wkT = jnp.concatenate([hp['Wk'].T for hp in heads], axis=1)
        wvT = jnp.concatenate([hp['Wv'].T for hp in heads], axis=1)
        bq = jnp.concatenate([hp['bq'] for hp in heads]).reshape(1, -1)
        bk = jnp.concatenate([hp['bk'] for hp in heads]).reshape(1, -1)
        bv = jnp.concatenate([hp['bv'] for hp in heads]).reshape(1, -1)

        qt, kt, vt = _qkv(h, wqT, wkT, wvT, bq, bk, bv)
        prod = _sc_pass1(r3, c3, qt, kt)
        eexp = _logits(prod)
        accs = _sc_pass2(r2, c2, eexp, vt, zeros_acc)
        h = _post(h, accs[0], accs[1],
                  lp['lo_W'].T, lp['lo_b'].reshape(1, -1),
                  lp['f1_W'].T, lp['f1_b'].reshape(1, -1),
                  lp['f2_W'].T, lp['f2_b'].reshape(1, -1),
                  lp['ln1_g'].reshape(1, -1), lp['ln1_b'].reshape(1, -1),
                  lp['ln2_g'].reshape(1, -1), lp['ln2_b'].reshape(1, -1))

    return _final(h, params['out_W'].T, params['out_b'].reshape(1, -1))


# speedup vs baseline: 1.3679x; 1.0795x over previous
import functools

import jax
import jax.numpy as jnp
from jax import lax
from jax.experimental import pallas as pl
from jax.experimental.pallas import tpu as pltpu
from jax.experimental.pallas import tpu_sc as plsc

N = 10000
E = 320000
IN_CH = 128
HID = 144
OUT_CH = 128
HEADS = 9
HEAD_DIM = 16
FF = 4 * HID
ROW = 160
ROWQ = 256
SCALE = 0.25

NC = 2
NS = 16
NW = NC * NS
EW = E // NW
CB = 80
NCH = EW // CB
CB2 = 40
NCH2 = EW // CB2

BN = 1000
BE = 8000

_HI = lax.Precision.HIGHEST
_f32 = jnp.float32


def _dot(a, b):
    return jnp.dot(a, b, precision=_HI, preferred_element_type=_f32)


def _acos(x):
    ax = jnp.abs(x)
    p = jnp.float32(-0.0012624911)
    for c in (0.0066700901, -0.0170881256, 0.0308918810, -0.0501743046,
              0.0889789874, -0.2145988016, 1.5707963050):
        p = p * ax + jnp.float32(c)
    p = p * jnp.sqrt(jnp.maximum(1.0 - ax, 0.0))
    return jnp.where(x >= 0, p, jnp.float32(jnp.pi) - p)


def _lane_iota(n):
    return lax.broadcasted_iota(jnp.int32, (1, n), 1)


def _sel_chunk():
    r = lax.broadcasted_iota(jnp.int32, (HID, HEADS), 0)
    c = lax.broadcasted_iota(jnp.int32, (HID, HEADS), 1)
    return (r // HEAD_DIM == c).astype(_f32)


def _sel_bcast():
    r = lax.broadcasted_iota(jnp.int32, (HEADS, HID), 0)
    c = lax.broadcasted_iota(jnp.int32, (HEADS, HID), 1)
    return (c // HEAD_DIM == r).astype(_f32)


def _sel_block():
    r = lax.broadcasted_iota(jnp.int32, (HID, HID), 0)
    c = lax.broadcasted_iota(jnp.int32, (HID, HID), 1)
    return (r // HEAD_DIM == c // HEAD_DIM).astype(_f32)


def _ln(x, g, b):
    m = jnp.mean(x, axis=-1, keepdims=True)
    v = jnp.mean((x - m) * (x - m), axis=-1, keepdims=True)
    return (x - m) / jnp.sqrt(v + 1e-5) * g + b


def _emb_body(x_ref, w_ref, b_ref, o_ref):
    o_ref[...] = _dot(x_ref[...], w_ref[...]) + b_ref[...]


def _emb(x, wT, b):
    return pl.pallas_call(
        _emb_body,
        grid=(N // BN,),
        in_specs=[
            pl.BlockSpec((BN, IN_CH), lambda i: (i, 0)),
            pl.BlockSpec((IN_CH, HID), lambda i: (0, 0)),
            pl.BlockSpec((1, HID), lambda i: (0, 0)),
        ],
        out_specs=pl.BlockSpec((BN, HID), lambda i: (i, 0)),
        out_shape=jax.ShapeDtypeStruct((N, HID), _f32),
    )(x, wT, b)


def _qkv_body(h_ref, wq_ref, wk_ref, wv_ref, bq_ref, bk_ref, bv_ref,
              qt_ref, kt_ref, vt_ref):
    h = h_ref[...]
    B = h.shape[0]
    SS = _sel_block()
    S = _sel_chunk()
    l144 = _lane_iota(HID)

    def csum(x):
        return _dot(x, SS)

    def csum9(x):
        return _dot(x, S)

    n = jnp.sqrt(jnp.sum(h * h, axis=-1, keepdims=True) + 1e-12)
    un = jnp.clip(n, 1e-7, 1.0 - 1e-5)
    vh = (0.5 * jnp.log((1.0 + un) / (1.0 - un))) * h / jnp.maximum(n, 1e-7)
    xs = h / n
    d0 = jnp.clip(xs[:, HID - 1:HID], -1.0 + 1e-6, 1.0 - 1e-6)
    o143 = (l144 == HID - 1).astype(_f32)
    uu = xs - d0 * o143
    unorm = jnp.sqrt(jnp.sum(uu * uu, axis=-1, keepdims=True) + 1e-12)
    vs = _acos(d0) * uu / jnp.maximum(unorm, 1e-7)
    ve = h

    mhyp = l144 < 48
    meuc = (l144 >= 48) & (l144 < 96)
    o15 = (l144 % HEAD_DIM == HEAD_DIM - 1).astype(_f32)

    def mix(w, b):
        return jnp.concatenate(
            [_dot(vh, w[:, 0:48]), _dot(ve, w[:, 48:96]),
             _dot(vs, w[:, 96:144])], axis=-1) + b

    def expmap(pre):
        cn = jnp.sqrt(csum(pre * pre) + 1e-12)
        yh = jnp.tanh(cn) * pre / jnp.maximum(cn, 1e-7)
        yn = jnp.sqrt(csum(yh * yh) + 1e-12)
        oh = yh * jnp.minimum(1.0, (1.0 - 1e-5) / jnp.maximum(yn, 1e-7))
        ys = jnp.cos(cn) * o15 + jnp.sin(cn) * pre / jnp.maximum(cn, 1e-7)
        osph = ys / jnp.sqrt(csum(ys * ys) + 1e-12)
        return jnp.where(mhyp, oh, jnp.where(meuc, pre, osph))

    q = expmap(mix(wq_ref[...], bq_ref[...]))
    k = expmap(mix(wk_ref[...], bk_ref[...]))
    v = expmap(mix(wv_ref[...], bv_ref[...]))

    nq3 = csum9(q * q)[:, 0:3]
    nk3 = csum9(k * k)[:, 0:3]
    one3 = jnp.ones((B, 3), _f32)

    rr = lax.broadcasted_iota(jnp.int32, (HID, ROWQ), 0)
    cc = lax.broadcasted_iota(jnp.int32, (HID, ROWQ), 1)
    P = (((cc % 16) == (rr // 16)) & ((cc // 16) == (rr % 16))).astype(_f32)

    def esel(off):
        r3 = lax.broadcasted_iota(jnp.int32, (3, ROWQ), 0)
        c3 = lax.broadcasted_iota(jnp.int32, (3, ROWQ), 1)
        return (c3 == off + r3).astype(_f32)

    e0 = esel(9)
    e1 = esel(25)
    e2 = esel(44)
    qt_ref[...] = (_dot(q, P) + _dot(nq3, e0) + _dot(one3, e1)
                   + _dot(1.0 - nq3, e2))
    kt_ref[...] = (_dot(k, P) + _dot(one3, e0) + _dot(nk3, e1)
                   + _dot(1.0 - nk3, e2))
    vt_ref[...] = jnp.concatenate(
        [v, jnp.ones((B, HEADS), _f32), jnp.zeros((B, 7), _f32)], axis=-1)


def _qkv(h, wqT, wkT, wvT, bq, bk, bv):
    wspec = pl.BlockSpec((HID, HID), lambda i: (0, 0))
    bspec = pl.BlockSpec((1, HID), lambda i: (0, 0))
    qspec = pl.BlockSpec((BN, ROWQ), lambda i: (i, 0))
    qshape = jax.ShapeDtypeStruct((N, ROWQ), _f32)
    return pl.pallas_call(
        _qkv_body,
        grid=(N // BN,),
        in_specs=[pl.BlockSpec((BN, HID), lambda i: (i, 0)),
                  wspec, wspec, wspec, bspec, bspec, bspec],
        out_specs=(qspec, qspec,
                   pl.BlockSpec((BN, ROW), lambda i: (i, 0))),
        out_shape=(qshape, qshape,
                   jax.ShapeDtypeStruct((N, ROW), _f32)),
    )(h, wqT, wkT, wvT, bq, bk, bv)


def _sc1_body(ridx3_ref, cidx3_ref, qt_ref, kt_ref, prod_ref,
              ridxv, cidxv, qb0, kb0, pb0, qb1, kb1, pb1,
              gsem0, gsem1, wsem0, wsem1):
    c = lax.axis_index("c")
    s = lax.axis_index("s")
    wid = s * NC + c
    base0 = wid * EW
    pltpu.sync_copy(ridx3_ref.at[wid], ridxv)
    pltpu.sync_copy(cidx3_ref.at[wid], cidxv)

    bufs = ((qb0, kb0, pb0, gsem0, wsem0), (qb1, kb1, pb1, gsem1, wsem1))

    def gpair(i, p):
        qb, kb, _, gsem, _ = bufs[p]
        gq = pltpu.make_async_copy(qt_ref.at[ridxv.at[i]], qb, gsem)
        gk = pltpu.make_async_copy(kt_ref.at[cidxv.at[i]], kb, gsem)
        return gq, gk

    def wb(i, p):
        _, _, pb, _, wsem = bufs[p]
        return pltpu.make_async_copy(
            pb, prod_ref.at[pl.ds(base0 + i * CB, CB)], wsem)

    for p in (0, 1):
        gq, gk = gpair(p, p)
        gq.start()
        gk.start()

    def chunk(i, carry):
        for p in (0, 1):

            @pl.when(i % 2 == p)
            def _():
                qb, kb, pb, gsem, wsem = bufs[p]
                gq, gk = gpair(i, p)
                gq.wait()
                gk.wait()

                @pl.when(i >= 2)
                def _():
                    wb(i - 2, p).wait()

                def mrow(j, carry2):
                    acc = qb[j, pl.ds(0, 16)] * kb[j, pl.ds(0, 16)]
                    for t in range(1, ROWQ // 16):
                        sl = pl.ds(t * 16, 16)
                        acc = acc + qb[j, sl] * kb[j, sl]
                    pb[j, pl.ds(0, 16)] = acc
                    return carry2

                lax.fori_loop(0, CB, mrow, 0)

                @pl.when(i + 2 < NCH)
                def _():
                    g2q, g2k = gpair(i + 2, p)
                    g2q.start()
                    g2k.start()

                wb(i, p).start()
        return carry

    lax.fori_loop(0, NCH, chunk, 0)
    wb(NCH - 2, NCH % 2).wait()
    wb(NCH - 1, (NCH - 1) % 2).wait()


def _sc_pass1(ridx3, cidx3, qt, kt):
    call = pl.kernel(
        _sc1_body,
        out_type=jax.ShapeDtypeStruct((E, 16), _f32),
        mesh=plsc.VectorSubcoreMesh(core_axis_name="c", subcore_axis_name="s",
                                    num_cores=NC, num_subcores=NS),
        compiler_params=pltpu.CompilerParams(use_tc_tiling_on_sc=False),
        scratch_types=[
            pltpu.VMEM((NCH, CB), jnp.int32),
            pltpu.VMEM((NCH, CB), jnp.int32),
            pltpu.VMEM((CB, ROWQ), _f32),
            pltpu.VMEM((CB, ROWQ), _f32),
            pltpu.VMEM((CB, 16), _f32),
            pltpu.VMEM((CB, ROWQ), _f32),
            pltpu.VMEM((CB, ROWQ), _f32),
            pltpu.VMEM((CB, 16), _f32),
            pltpu.SemaphoreType.DMA,
            pltpu.SemaphoreType.DMA,
            pltpu.SemaphoreType.DMA,
            pltpu.SemaphoreType.DMA,
        ],
    )
    return call(ridx3, cidx3, qt, kt)


def _logits_body(prod_ref, eexp_ref):
    prod = prod_ref[...]
    B = prod.shape[0]
    dots = prod[:, 0:HEADS]
    s3 = prod[:, 9:12]
    den3 = prod[:, 12:15]
    alpha = dots * SCALE
    d2 = s3 - 2.0 * dots[:, 0:3]
    den = jnp.maximum(den3, 1e-7)
    z = jnp.maximum(1.0 + 2.0 * d2 / den, 1.0 + 1e-7)
    dh = jnp.log(z + jnp.sqrt(z * z - 1.0))
    t = jnp.clip(dots[:, 6:9], -1.0 + 1e-6, 1.0 - 1e-6)
    corr = jnp.concatenate(
        [0.1 * dh, jnp.zeros((B, 3), _f32), 0.1 * _acos(t)], axis=-1)
    e9 = jnp.exp(alpha - corr)
    e144 = _dot(e9, _sel_bcast())
    eexp_ref[...] = jnp.concatenate(
        [e144, e9, jnp.zeros((B, 7), _f32)], axis=-1)


def _logits(prod):
    return pl.pallas_call(
        _logits_body,
        grid=(E // BE,),
        in_specs=[pl.BlockSpec((BE, 16), lambda i: (i, 0))],
        out_specs=pl.BlockSpec((BE, ROW), lambda i: (i, 0)),
        out_shape=jax.ShapeDtypeStruct((E, ROW), _f32),
    )(prod)


def _sc2_body(ridx2_ref, cidx2_ref, eexp_ref, vt_ref, zero_ref, out_ref,
              cb0, cb1, rb0, rb1, vb0, vb1, eb0, eb1, acc_sh,
              csem0, csem1, rsem0, rsem1, gsem0, gsem1, esem0, esem1,
              ssem0, ssem1):
    c = lax.axis_index("c")
    s = lax.axis_index("s")
    wid = s * NC + c
    base0 = wid * EW

    @pl.when(s == 0)
    def _zero():
        pltpu.sync_copy(zero_ref, acc_sh)

    plsc.subcore_barrier()

    cbufs = (cb0, cb1)
    rbufs = (rb0, rb1)
    vbufs = (vb0, vb1)
    ebufs = (eb0, eb1)
    csems = (csem0, csem1)
    rsems = (rsem0, rsem1)
    gsems = (gsem0, gsem1)
    esems = (esem0, esem1)
    ssems = (ssem0, ssem1)

    def cidx_cp(i, p):
        return pltpu.make_async_copy(
            cidx2_ref.at[wid * NCH2 + i], cbufs[p], csems[p])

    def ridx_cp(i, p):
        return pltpu.make_async_copy(
            ridx2_ref.at[wid * NCH2 + i], rbufs[p], rsems[p])

    def eexp_cp(i, p):
        return pltpu.make_async_copy(
            eexp_ref.at[pl.ds(base0 + i * CB2, CB2)], ebufs[p], esems[p])

    def gath(p):
        return pltpu.make_async_copy(vt_ref.at[cbufs[p]], vbufs[p], gsems[p])

    def scat(p):
        return pltpu.make_async_copy(vbufs[p], acc_sh.at[rbufs[p]], ssems[p])

    cidx_cp(0, 0).start()
    cidx_cp(1, 1).start()
    ridx_cp(0, 0).start()
    eexp_cp(0, 0).start()
    eexp_cp(1, 1).start()
    cidx_cp(0, 0).wait()
    gath(0).start()

    def chunk(i, carry):
        for p in (0, 1):

            @pl.when(i % 2 == p)
            def _():
                gath(p).wait()
                eexp_cp(i, p).wait()

                def mrow(j, carry2):
                    for t in range(ROW // 16):
                        sl = pl.ds(t * 16, 16)
                        vbufs[p][j, sl] = ebufs[p][j, sl] * vbufs[p][j, sl]
                    return carry2

                lax.fori_loop(0, CB2, mrow, 0)

                @pl.when(i + 2 < NCH2)
                def _():
                    eexp_cp(i + 2, p).start()

                ridx_cp(i, p).wait()
                pltpu.async_copy(vbufs[p], acc_sh.at[rbufs[p]],
                                 ssems[p], add=True)

                @pl.when(i >= 1)
                def _():
                    scat(1 - p).wait()

                @pl.when(i + 1 < NCH2)
                def _():
                    cidx_cp(i + 1, 1 - p).wait()
                    gath(1 - p).start()
                    ridx_cp(i + 1, 1 - p).start()

                @pl.when(i + 2 < NCH2)
                def _():
                    cidx_cp(i + 2, p).start()
        return carry

    lax.fori_loop(0, NCH2, chunk, 0)
    scat((NCH2 - 1) % 2).wait()
    plsc.subcore_barrier()

    @pl.when(s == 0)
    def _flush():
        pltpu.sync_copy(acc_sh, out_ref.at[c])


def _sc_pass2(ridx2, cidx2, eexp, vt, zeros_acc):
    call = pl.kernel(
        _sc2_body,
        out_type=jax.ShapeDtypeStruct((NC, N, ROW), _f32),
        mesh=plsc.VectorSubcoreMesh(core_axis_name="c", subcore_axis_name="s",
                                    num_cores=NC, num_subcores=NS),
        compiler_params=pltpu.CompilerParams(use_tc_tiling_on_sc=False),
        scratch_types=[
            pltpu.VMEM((CB2,), jnp.int32),
            pltpu.VMEM((CB2,), jnp.int32),
            pltpu.VMEM((CB2,), jnp.int32),
            pltpu.VMEM((CB2,), jnp.int32),
            pltpu.VMEM((CB2, ROW), _f32),
            pltpu.VMEM((CB2, ROW), _f32),
            pltpu.VMEM((CB2, ROW), _f32),
            pltpu.VMEM((CB2, ROW), _f32),
            pltpu.VMEM_SHARED((N, ROW), _f32),
            pltpu.SemaphoreType.DMA,
            pltpu.SemaphoreType.DMA,
            pltpu.SemaphoreType.DMA,
            pltpu.SemaphoreType.DMA,
            pltpu.SemaphoreType.DMA,
            pltpu.SemaphoreType.DMA,
            pltpu.SemaphoreType.DMA,
            pltpu.SemaphoreType.DMA,
            pltpu.SemaphoreType.DMA,
            pltpu.SemaphoreType.DMA,
        ],
    )
    return call(ridx2, cidx2, eexp, vt, zeros_acc)


def _post_body(h_ref, a0_ref, a1_ref, lo_ref, lob_ref, f1_ref, f1b_ref,
               f2_ref, f2b_ref, g1_ref, b1_ref, g2_ref, b2_ref, out_ref):
    acc = a0_ref[...] + a1_ref[...]
    s144 = _dot(acc[:, 144:144 + HEADS], _sel_bcast())
    att = acc[:, 0:HID] / (s144 + 1e-16)
    att = _dot(att, lo_ref[...]) + lob_ref[...]
    h1 = _ln(h_ref[...] + att, g1_ref[...], b1_ref[...])
    ffp = _dot(h1, f1_ref[...]) + f1b_ref[...]
    gl = ffp * 0.5 * (1.0 + lax.erf(ffp * jnp.float32(0.7071067811865475)))
    ff = _dot(gl, f2_ref[...]) + f2b_ref[...]
    out_ref[...] = _ln(h1 + ff, g2_ref[...], b2_ref[...])


def _post(h, a0, a1, loT, lob, f1T, f1b, f2T, f2b, g1, b1, g2, b2):
    vspec = pl.BlockSpec((1, HID), lambda i: (0, 0))
    return pl.pallas_call(
        _post_body,
        grid=(N // BN,),
        in_specs=[
            pl.BlockSpec((BN, HID), lambda i: (i, 0)),
            pl.BlockSpec((BN, ROW), lambda i: (i, 0)),
            pl.BlockSpec((BN, ROW), lambda i: (i, 0)),
            pl.BlockSpec((HID, HID), lambda i: (0, 0)), vspec,
            pl.BlockSpec((HID, FF), lambda i: (0, 0)),
            pl.BlockSpec((1, FF), lambda i: (0, 0)),
            pl.BlockSpec((FF, HID), lambda i: (0, 0)), vspec,
            vspec, vspec, vspec, vspec,
        ],
        out_specs=pl.BlockSpec((BN, HID), lambda i: (i, 0)),
        out_shape=jax.ShapeDtypeStruct((N, HID), _f32),
    )(h, a0, a1, loT, lob, f1T, f1b, f2T, f2b, g1, b1, g2, b2)


def _final_body(h_ref, w_ref, b_ref, o_ref):
    o_ref[...] = _dot(h_ref[...], w_ref[...]) + b_ref[...]


def _final(h, wT, b):
    return pl.pallas_call(
        _final_body,
        grid=(N // BN,),
        in_specs=[
            pl.BlockSpec((BN, HID), lambda i: (i, 0)),
            pl.BlockSpec((HID, OUT_CH), lambda i: (0, 0)),
            pl.BlockSpec((1, OUT_CH), lambda i: (0, 0)),
        ],
        out_specs=pl.BlockSpec((BN, OUT_CH), lambda i: (i, 0)),
        out_shape=jax.ShapeDtypeStruct((N, OUT_CH), _f32),
    )(h, wT, b)


def kernel(x, edge_index, params):
    row = edge_index[0]
    col = edge_index[1]
    r3 = row.reshape(NW, NCH, CB)
    c3 = col.reshape(NW, NCH, CB)
    r2 = row.reshape(NW * NCH2, CB2)
    c2 = col.reshape(NW * NCH2, CB2)
    zeros_acc = jnp.zeros((N, ROW), _f32)

    h = _emb(x, params['emb_W'].T, params['emb_b'].reshape(1, -1))
    for lp in params['layers']:
        heads = lp['heads']
        wqT = jnp.concatenate([hp['Wq'].T for hp in heads], axis=1)
        wkT = jnp.concatenate([hp['Wk'].T for hp in heads], axis=1)
        wvT = jnp.concatenate([hp['Wv'].T for hp in heads], axis=1)
        bq = jnp.concatenate([hp['bq'] for hp in heads]).reshape(1, -1)
        bk = jnp.concatenate([hp['bk'] for hp in heads]).reshape(1, -1)
        bv = jnp.concatenate([hp['bv'] for hp in heads]).reshape(1, -1)

        qt, kt, vt = _qkv(h, wqT, wkT, wvT, bq, bk, bv)
        prod = _sc_pass1(r3, c3, qt, kt)
        eexp = _logits(prod)
        accs = _sc_pass2(r2, c2, eexp, vt, zeros_acc)
        h = _post(h, accs[0], accs[1],
                  lp['lo_W'].T, lp['lo_b'].reshape(1, -1),
                  lp['f1_W'].T, lp['f1_b'].reshape(1, -1),
                  lp['f2_W'].T, lp['f2_b'].reshape(1, -1),
                  lp['ln1_g'].reshape(1, -1), lp['ln1_b'].reshape(1, -1),
                  lp['ln2_g'].reshape(1, -1), lp['ln2_b'].reshape(1, -1))

    return _final(h, params['out_W'].T, params['out_b'].reshape(1, -1))
